# baseline (speedup 1.0000x reference)
import jax
import jax.numpy as jnp
from jax.experimental import pallas as pl
from jax.experimental.pallas import tpu as pltpu

N = 1024
R = 4
IN_DIM = 256
HID = 256
BN = 256
NB = N // BN


def _body(adj_ref, xh_ref, w1_ref, w2_ref, h_ref, out_ref, z_s, t_s, g_s):
    p = pl.program_id(1)
    i = pl.program_id(2)
    a = adj_ref[0]

    @pl.when(p == 0)
    def _phase0():
        axh = jnp.dot(a, xh_ref[0], preferred_element_type=jnp.float32)
        pre = jnp.dot(axh, w1_ref[0], preferred_element_type=jnp.float32)
        z = jax.nn.sigmoid(pre[:, :HID])
        rg = jax.nn.sigmoid(pre[:, HID:2 * HID])
        z_s[pl.ds(i * BN, BN), :] = z
        t_s[pl.ds(i * BN, BN), :] = pre[:, 2 * HID:]
        g_s[pl.ds(i * BN, BN), :] = rg * h_ref[0]
        out_ref[0, 0] = jnp.zeros((BN, HID), jnp.float32)

    @pl.when(p == 1)
    def _phase1():
        ag = jnp.dot(a, g_s[:, :], preferred_element_type=jnp.float32)
        ht = jnp.tanh(t_s[pl.ds(i * BN, BN), :]
                      + jnp.dot(ag, w2_ref[0], preferred_element_type=jnp.float32))
        z = z_s[pl.ds(i * BN, BN), :]
        out_ref[0, 0] = z * h_ref[0] + (1.0 - z) * ht


def kernel(X, adj, h_pre, W_xz, W_xr, W_xh, W_hz, W_hr, W_hh):
    del W_hh
    Xr = jnp.transpose(X, (2, 0, 1))
    Hr = jnp.transpose(h_pre, (2, 0, 1))
    XH = jnp.concatenate([Xr, Hr], axis=2)
    W_top = jnp.concatenate([W_xz, W_xr, W_xh], axis=2)
    W_bot = jnp.concatenate([W_hz, W_hr, jnp.zeros_like(W_hr)], axis=2)
    W1 = jnp.concatenate([W_top, W_bot], axis=1)

    out = pl.pallas_call(
        _body,
        grid=(R, 2, NB),
        in_specs=[
            pl.BlockSpec((1, BN, N), lambda r, p, i: (r, i, 0)),
            pl.BlockSpec((1, N, IN_DIM + HID), lambda r, p, i: (r, 0, 0)),
            pl.BlockSpec((1, IN_DIM + HID, 3 * HID), lambda r, p, i: (r, 0, 0)),
            pl.BlockSpec((1, HID, HID), lambda r, p, i: (r, 0, 0)),
            pl.BlockSpec((1, BN, HID), lambda r, p, i: (r, i, 0)),
        ],
        out_specs=pl.BlockSpec((1, 1, BN, HID), lambda r, p, i: (p, r, i, 0)),
        out_shape=jax.ShapeDtypeStruct((2, R, N, HID), jnp.float32),
        scratch_shapes=[
            pltpu.VMEM((N, HID), jnp.float32),
            pltpu.VMEM((N, HID), jnp.float32),
            pltpu.VMEM((N, HID), jnp.float32),
        ],
        compiler_params=pltpu.CompilerParams(
            dimension_semantics=("arbitrary", "arbitrary", "arbitrary"),
        ),
    )(adj, XH, W1, W_hr, Hr)

    return jnp.transpose(out[1], (1, 2, 0))
